# baseline (speedup 1.0000x reference)
import functools

import numpy as np

import jax
import jax.numpy as jnp
from jax import lax
from jax.experimental import pallas as pl
from jax.experimental.pallas import tpu as pltpu
from jax.experimental.pallas import tpu_sc as plsc

HIDDEN = 128
PAD = 0
EPS = 1e-5

NC = 2
NS = 16
NW = NC * NS
L = 16
NBLK = HIDDEN // L

C = 128

_RSQRT_MAGIC = np.int32(0x5F3759DF)


def _rsqrt(a):
    ai = lax.bitcast_convert_type(a, jnp.int32)
    y = lax.bitcast_convert_type(_RSQRT_MAGIC - (ai >> 1), jnp.float32)
    ha = a * 0.5
    for _ in range(3):
        y = y * (1.5 - ha * y * y)
    return y


def _make_kernel(n_tokens):
    assert n_tokens % (NW * C) == 0
    per_w = n_tokens // NW
    n_chunks = per_w // C
    assert n_chunks % 2 == 0 and n_chunks >= 4

    mesh = plsc.VectorSubcoreMesh(
        core_axis_name="c", subcore_axis_name="s",
        num_cores=NC, num_subcores=NS,
    )

    @functools.partial(
        pl.kernel,
        out_type=(
            jax.ShapeDtypeStruct((n_tokens, HIDDEN), jnp.float32),
            jax.ShapeDtypeStruct((n_tokens,), jnp.int32),
        ),
        mesh=mesh,
        compiler_params=pltpu.CompilerParams(needs_layout_passes=False),
        scratch_types=[
            pltpu.VMEM((2, C), jnp.int32),
            pltpu.VMEM((2, C), jnp.int32),
            pltpu.VMEM((2, C, HIDDEN), jnp.float32),
            pltpu.VMEM((C, HIDDEN), jnp.float32),
            pltpu.VMEM((C,), jnp.int32),
            pltpu.VMEM((HIDDEN,), jnp.float32),
            pltpu.VMEM((HIDDEN,), jnp.float32),
            pltpu.SemaphoreType.DMA,
            pltpu.SemaphoreType.DMA,
            pltpu.SemaphoreType.DMA,
            pltpu.SemaphoreType.DMA,
        ],
    )
    def emb_kernel(idw_hbm, idp_hbm, wtab_hbm, ptab_hbm, g_hbm, b_hbm,
                   out_hbm, mask_hbm,
                   idw_v, idp_v, xrows, orows, mvec, gv, bv,
                   semw0, semw1, semp0, semp1):
        wid = lax.axis_index("s") * NC + lax.axis_index("c")
        base = wid * per_w

        pltpu.sync_copy(g_hbm, gv)
        pltpu.sync_copy(b_hbm, bv)
        gs = [gv[pl.ds(L * e, L)] for e in range(NBLK)]
        bs = [bv[pl.ds(L * e, L)] for e in range(NBLK)]
        semw = (semw0, semw1)
        semp = (semp0, semp1)

        def copy_ids(ci, slot):
            off = base + ci * C
            pltpu.sync_copy(idw_hbm.at[pl.ds(off, C)], idw_v.at[slot])
            pltpu.sync_copy(idp_hbm.at[pl.ds(off, C)], idp_v.at[slot])

        def issue_w(slot):
            pltpu.async_copy(wtab_hbm.at[idw_v.at[slot]],
                             xrows.at[slot], semw[slot])

        def wait_w(slot):
            pltpu.make_async_copy(wtab_hbm.at[idw_v.at[slot]],
                                  xrows.at[slot], semw[slot]).wait()

        def issue_p(slot):
            pltpu.async_copy(ptab_hbm.at[idp_v.at[slot]],
                             xrows.at[slot], semp[slot], add=True)

        def wait_p(slot):
            pltpu.make_async_copy(ptab_hbm.at[idp_v.at[slot]],
                                  xrows.at[slot], semp[slot]).wait()

        def ln_half(slot, lo):
            rows = xrows.at[slot]

            def tok_body(t, c2):
                xs = [rows[t, pl.ds(L * e, L)] for e in range(NBLK)]
                s = xs[0]
                ssq = xs[0] * xs[0]
                for e in range(1, NBLK):
                    s = s + xs[e]
                    ssq = ssq + xs[e] * xs[e]
                mean = jnp.sum(s) * (1.0 / HIDDEN)
                var = jnp.sum(ssq) * (1.0 / HIDDEN) - mean * mean
                inv = _rsqrt(var + EPS)
                for e in range(NBLK):
                    t1 = gs[e] * inv
                    orows[t, pl.ds(L * e, L)] = (xs[e] - mean) * t1 + bs[e]
                return c2

            lax.fori_loop(lo, lo + C // 2, tok_body, 0, unroll=2)

        def mask_pass(slot):
            idws = idw_v.at[slot]

            def mask_body(g, c2):
                v = idws[pl.ds(g * L, L)]
                mvec[pl.ds(g * L, L)] = 1 - jnp.minimum(v, 1)
                return c2

            lax.fori_loop(0, C // L, mask_body, 0)

        def store(ci):
            off = base + ci * C
            pltpu.sync_copy(orows, out_hbm.at[pl.ds(off, C)])
            pltpu.sync_copy(mvec, mask_hbm.at[pl.ds(off, C)])

        def do_chunk(ci, slot, nxt):
            other = 1 - slot
            wait_p(slot)
            mask_pass(slot)
            ln_half(slot, 0)
            if nxt is not None:
                wait_w(other)
                issue_p(other)
            ln_half(slot, C // 2)
            store(ci)
            if nxt is not None:
                copy_ids(nxt, slot)
                issue_w(slot)

        copy_ids(0, 0)
        copy_ids(1, 1)
        issue_w(0)
        wait_w(0)
        issue_p(0)
        issue_w(1)

        def pair_body(k, carry):
            i0 = 2 * k
            do_chunk(i0, 0, i0 + 2)
            do_chunk(i0 + 1, 1, i0 + 3)
            return carry

        lax.fori_loop(0, n_chunks // 2 - 1, pair_body, 0)

        wait_p(0)
        mask_pass(0)
        ln_half(0, 0)
        wait_w(1)
        issue_p(1)
        ln_half(0, C // 2)
        store(n_chunks - 2)
        wait_p(1)
        mask_pass(1)
        ln_half(1, 0)
        ln_half(1, C // 2)
        store(n_chunks - 1)

    return emb_kernel


@jax.jit
def _run(idw, idp, word_emb, pos_emb, ln_gamma, ln_beta):
    n_tokens = idw.shape[0]
    w = word_emb.at[PAD].set(0.0)
    p = pos_emb.at[PAD].set(0.0)
    return _make_kernel(n_tokens)(idw, idp, w, p, ln_gamma, ln_beta)


def kernel(uttr_ids_list, position_ids_list, word_emb, pos_emb, ln_gamma,
           ln_beta):
    B, S = uttr_ids_list.shape
    n = B * S
    out, mask = _run(uttr_ids_list.reshape(n), position_ids_list.reshape(n),
                     word_emb, pos_emb, ln_gamma, ln_beta)
    return out.reshape(B, S, HIDDEN), mask.reshape(B, S).astype(bool)


# speedup vs baseline: 1.0426x; 1.0426x over previous
import functools

import numpy as np

import jax
import jax.numpy as jnp
from jax import lax
from jax.experimental import pallas as pl
from jax.experimental.pallas import tpu as pltpu
from jax.experimental.pallas import tpu_sc as plsc

HIDDEN = 128
PAD = 0
EPS = 1e-5

NC = 2
NS = 16
NW = NC * NS
L = 16
NBLK = HIDDEN // L

C = 128

_RSQRT_MAGIC = np.int32(0x5F3759DF)


_GATHER_DNUMS = lax.GatherDimensionNumbers(
    offset_dims=(), collapsed_slice_dims=(0,), start_index_map=(0,))


def _splat(v, j):
    idx = jnp.full((L, 1), j, jnp.int32)
    return lax.gather(v, idx, _GATHER_DNUMS, (1,),
                      mode=lax.GatherScatterMode.PROMISE_IN_BOUNDS)


def _rsqrt(a):
    ai = lax.bitcast_convert_type(a, jnp.int32)
    y = lax.bitcast_convert_type(_RSQRT_MAGIC - (ai >> 1), jnp.float32)
    ha = a * 0.5
    for _ in range(3):
        y = y * (1.5 - ha * y * y)
    return y


def _make_kernel(n_tokens):
    assert n_tokens % (NW * C) == 0
    per_w = n_tokens // NW
    n_chunks = per_w // C
    assert n_chunks % 2 == 0 and n_chunks >= 4

    mesh = plsc.VectorSubcoreMesh(
        core_axis_name="c", subcore_axis_name="s",
        num_cores=NC, num_subcores=NS,
    )

    @functools.partial(
        pl.kernel,
        out_type=(
            jax.ShapeDtypeStruct((n_tokens, HIDDEN), jnp.float32),
            jax.ShapeDtypeStruct((n_tokens,), jnp.int32),
        ),
        mesh=mesh,
        compiler_params=pltpu.CompilerParams(needs_layout_passes=False),
        scratch_types=[
            pltpu.VMEM((2, C), jnp.int32),
            pltpu.VMEM((2, C), jnp.int32),
            pltpu.VMEM((2, C, HIDDEN), jnp.float32),
            pltpu.VMEM((C, HIDDEN), jnp.float32),
            pltpu.VMEM((17 * L,), jnp.float32),
            pltpu.VMEM((17 * L,), jnp.float32),
            pltpu.VMEM((C,), jnp.int32),
            pltpu.VMEM((HIDDEN,), jnp.float32),
            pltpu.VMEM((HIDDEN,), jnp.float32),
            pltpu.SemaphoreType.DMA,
            pltpu.SemaphoreType.DMA,
            pltpu.SemaphoreType.DMA,
            pltpu.SemaphoreType.DMA,
        ],
    )
    def emb_kernel(idw_hbm, idp_hbm, wtab_hbm, ptab_hbm, g_hbm, b_hbm,
                   out_hbm, mask_hbm,
                   idw_v, idp_v, xrows, orows, msum, msq, mvec, gv, bv,
                   semw0, semw1, semp0, semp1):
        wid = lax.axis_index("s") * NC + lax.axis_index("c")
        base = wid * per_w

        pltpu.sync_copy(g_hbm, gv)
        pltpu.sync_copy(b_hbm, bv)
        gs = [gv[pl.ds(L * e, L)] for e in range(NBLK)]
        bs = [bv[pl.ds(L * e, L)] for e in range(NBLK)]
        semw = (semw0, semw1)
        semp = (semp0, semp1)

        def copy_ids(ci, slot):
            off = base + ci * C
            pltpu.sync_copy(idw_hbm.at[pl.ds(off, C)], idw_v.at[slot])
            pltpu.sync_copy(idp_hbm.at[pl.ds(off, C)], idp_v.at[slot])

        def issue_w(slot):
            pltpu.async_copy(wtab_hbm.at[idw_v.at[slot]],
                             xrows.at[slot], semw[slot])

        def wait_w(slot):
            pltpu.make_async_copy(wtab_hbm.at[idw_v.at[slot]],
                                  xrows.at[slot], semw[slot]).wait()

        def issue_p(slot):
            pltpu.async_copy(ptab_hbm.at[idp_v.at[slot]],
                             xrows.at[slot], semp[slot], add=True)

        def wait_p(slot):
            pltpu.make_async_copy(ptab_hbm.at[idp_v.at[slot]],
                                  xrows.at[slot], semp[slot]).wait()

        def ln_half(slot, lo):
            rows = xrows.at[slot]
            i16 = lax.iota(jnp.int32, L)
            i17 = (i16 << 4) + i16

            def group_body(g, c2):
                gb = lo + g * L
                for tt in range(L):
                    t = gb + tt
                    xs = [rows[t, pl.ds(L * e, L)] for e in range(NBLK)]
                    s = xs[0]
                    ssq = xs[0] * xs[0]
                    for e in range(1, NBLK):
                        s = s + xs[e]
                        ssq = ssq + xs[e] * xs[e]
                    msum[pl.ds(17 * tt, L)] = s
                    msq[pl.ds(17 * tt, L)] = ssq
                tot = plsc.load_gather(msum, [i17])
                tot2 = plsc.load_gather(msq, [i17])
                for l in range(1, L):
                    tot = tot + plsc.load_gather(msum, [i17 + l])
                    tot2 = tot2 + plsc.load_gather(msq, [i17 + l])
                mean = tot * (1.0 / HIDDEN)
                var = tot2 * (1.0 / HIDDEN) - mean * mean
                inv = _rsqrt(var + EPS)
                for tt in range(L):
                    t = gb + tt
                    mv = _splat(mean, tt)
                    iv = _splat(inv, tt)
                    for e in range(NBLK):
                        t1 = gs[e] * iv
                        orows[t, pl.ds(L * e, L)] = (
                            (rows[t, pl.ds(L * e, L)] - mv) * t1 + bs[e])
                return c2

            lax.fori_loop(0, C // 2 // L, group_body, 0)

        def mask_pass(slot):
            idws = idw_v.at[slot]

            def mask_body(g, c2):
                v = idws[pl.ds(g * L, L)]
                mvec[pl.ds(g * L, L)] = 1 - jnp.minimum(v, 1)
                return c2

            lax.fori_loop(0, C // L, mask_body, 0)

        def store(ci):
            off = base + ci * C
            pltpu.sync_copy(orows, out_hbm.at[pl.ds(off, C)])
            pltpu.sync_copy(mvec, mask_hbm.at[pl.ds(off, C)])

        def do_chunk(ci, slot, nxt):
            other = 1 - slot
            wait_p(slot)
            mask_pass(slot)
            ln_half(slot, 0)
            if nxt is not None:
                wait_w(other)
                issue_p(other)
            ln_half(slot, C // 2)
            store(ci)
            if nxt is not None:
                copy_ids(nxt, slot)
                issue_w(slot)

        copy_ids(0, 0)
        copy_ids(1, 1)
        issue_w(0)
        wait_w(0)
        issue_p(0)
        issue_w(1)

        def pair_body(k, carry):
            i0 = 2 * k
            do_chunk(i0, 0, i0 + 2)
            do_chunk(i0 + 1, 1, i0 + 3)
            return carry

        lax.fori_loop(0, n_chunks // 2 - 1, pair_body, 0)

        wait_p(0)
        mask_pass(0)
        ln_half(0, 0)
        wait_w(1)
        issue_p(1)
        ln_half(0, C // 2)
        store(n_chunks - 2)
        wait_p(1)
        mask_pass(1)
        ln_half(1, 0)
        ln_half(1, C // 2)
        store(n_chunks - 1)

    return emb_kernel


@jax.jit
def _run(idw, idp, word_emb, pos_emb, ln_gamma, ln_beta):
    n_tokens = idw.shape[0]
    w = word_emb.at[PAD].set(0.0)
    p = pos_emb.at[PAD].set(0.0)
    return _make_kernel(n_tokens)(idw, idp, w, p, ln_gamma, ln_beta)


def kernel(uttr_ids_list, position_ids_list, word_emb, pos_emb, ln_gamma,
           ln_beta):
    B, S = uttr_ids_list.shape
    n = B * S
    out, mask = _run(uttr_ids_list.reshape(n), position_ids_list.reshape(n),
                     word_emb, pos_emb, ln_gamma, ln_beta)
    return out.reshape(B, S, HIDDEN), mask.reshape(B, S).astype(bool)


# speedup vs baseline: 1.4266x; 1.3683x over previous
import functools

import numpy as np

import jax
import jax.numpy as jnp
from jax import lax
from jax.experimental import pallas as pl
from jax.experimental.pallas import tpu as pltpu
from jax.experimental.pallas import tpu_sc as plsc

HIDDEN = 128
PAD = 0
EPS = 1e-5

NC = 2
NS = 16
NW = NC * NS
L = 16
NBLK = HIDDEN // L

C = 128

_RSQRT_MAGIC = np.int32(0x5F3759DF)

_GATHER_DNUMS = lax.GatherDimensionNumbers(
    offset_dims=(), collapsed_slice_dims=(0,), start_index_map=(0,))


def _splat(v, j):
    idx = jnp.full((L, 1), j, jnp.int32)
    return lax.gather(v, idx, _GATHER_DNUMS, (1,),
                      mode=lax.GatherScatterMode.PROMISE_IN_BOUNDS)


def _rsqrt(a):
    ai = lax.bitcast_convert_type(a, jnp.int32)
    y = lax.bitcast_convert_type(_RSQRT_MAGIC - (ai >> 1), jnp.float32)
    ha = a * 0.5
    for _ in range(3):
        y = y * (1.5 - ha * y * y)
    return y


def _make_kernel(n_tokens):
    assert n_tokens % (NW * C) == 0
    per_w = n_tokens // NW
    n_chunks = per_w // C
    assert n_chunks % 2 == 0 and n_chunks >= 4

    mesh = plsc.VectorSubcoreMesh(
        core_axis_name="c", subcore_axis_name="s",
        num_cores=NC, num_subcores=NS,
    )

    @functools.partial(
        pl.kernel,
        out_type=(
            jax.ShapeDtypeStruct((n_tokens, HIDDEN), jnp.float32),
            jax.ShapeDtypeStruct((n_tokens,), jnp.int32),
        ),
        mesh=mesh,
        compiler_params=pltpu.CompilerParams(needs_layout_passes=False),
        scratch_types=[
            pltpu.VMEM((per_w,), jnp.int32),
            pltpu.VMEM((per_w,), jnp.int32),
            pltpu.VMEM((2, C, HIDDEN), jnp.float32),
            pltpu.VMEM((2, C, HIDDEN), jnp.float32),
            pltpu.VMEM((17 * L,), jnp.float32),
            pltpu.VMEM((17 * L,), jnp.float32),
            pltpu.VMEM((2, C), jnp.int32),
            pltpu.VMEM((HIDDEN,), jnp.float32),
            pltpu.VMEM((HIDDEN,), jnp.float32),
            pltpu.SemaphoreType.DMA,
            pltpu.SemaphoreType.DMA,
            pltpu.SemaphoreType.DMA,
            pltpu.SemaphoreType.DMA,
            pltpu.SemaphoreType.DMA,
            pltpu.SemaphoreType.DMA,
        ],
    )
    def emb_kernel(idw_hbm, idp_hbm, wtab_hbm, ptab_hbm, g_hbm, b_hbm,
                   out_hbm, mask_hbm,
                   idw_v, idp_v, xrows, orows, msum, msq, mvec, gv, bv,
                   semw0, semw1, semp0, semp1, semo0, semo1):
        wid = lax.axis_index("s") * NC + lax.axis_index("c")
        base = wid * per_w

        pltpu.sync_copy(idw_hbm.at[pl.ds(base, per_w)], idw_v)
        pltpu.sync_copy(idp_hbm.at[pl.ds(base, per_w)], idp_v)
        pltpu.sync_copy(g_hbm, gv)
        pltpu.sync_copy(b_hbm, bv)
        gs = [gv[pl.ds(L * e, L)] for e in range(NBLK)]
        bs = [bv[pl.ds(L * e, L)] for e in range(NBLK)]
        semw = (semw0, semw1)
        semp = (semp0, semp1)
        semo = (semo0, semo1)

        def issue_w(ci, slot):
            pltpu.async_copy(wtab_hbm.at[idw_v.at[pl.ds(ci * C, C)]],
                             xrows.at[slot], semw[slot])

        def wait_w(ci, slot):
            pltpu.make_async_copy(wtab_hbm.at[idw_v.at[pl.ds(ci * C, C)]],
                                  xrows.at[slot], semw[slot]).wait()

        def issue_p(ci, slot):
            pltpu.async_copy(ptab_hbm.at[idp_v.at[pl.ds(ci * C, C)]],
                             xrows.at[slot], semp[slot], add=True)

        def wait_p(ci, slot):
            pltpu.make_async_copy(ptab_hbm.at[idp_v.at[pl.ds(ci * C, C)]],
                                  xrows.at[slot], semp[slot]).wait()

        def issue_store(ci, slot):
            off = base + ci * C
            pltpu.async_copy(orows.at[slot], out_hbm.at[pl.ds(off, C)],
                             semo[slot])
            pltpu.async_copy(mvec.at[slot], mask_hbm.at[pl.ds(off, C)],
                             semo[slot])

        def wait_store(ci, slot):
            off = base + ci * C
            pltpu.make_async_copy(orows.at[slot],
                                  out_hbm.at[pl.ds(off, C)],
                                  semo[slot]).wait()
            pltpu.make_async_copy(mvec.at[slot],
                                  mask_hbm.at[pl.ds(off, C)],
                                  semo[slot]).wait()

        def mask_pass(ci, slot):
            mv = mvec.at[slot]

            def mask_body(g, c2):
                v = idw_v[pl.ds(ci * C + g * L, L)]
                mv[pl.ds(g * L, L)] = 1 - jnp.minimum(v, 1)
                return c2

            lax.fori_loop(0, C // L, mask_body, 0)

        def ln_half(slot, lo):
            rows = xrows.at[slot]
            orws = orows.at[slot]
            i16 = lax.iota(jnp.int32, L)
            i17 = (i16 << 4) + i16

            def group_body(g, c2):
                gb = lo + g * L
                for tt in range(L):
                    t = gb + tt
                    xs = [rows[t, pl.ds(L * e, L)] for e in range(NBLK)]
                    s = xs[0]
                    ssq = xs[0] * xs[0]
                    for e in range(1, NBLK):
                        s = s + xs[e]
                        ssq = ssq + xs[e] * xs[e]
                    msum[pl.ds(17 * tt, L)] = s
                    msq[pl.ds(17 * tt, L)] = ssq
                tot = plsc.load_gather(msum, [i17])
                tot2 = plsc.load_gather(msq, [i17])
                for l in range(1, L):
                    tot = tot + plsc.load_gather(msum, [i17 + l])
                    tot2 = tot2 + plsc.load_gather(msq, [i17 + l])
                mean = tot * (1.0 / HIDDEN)
                var = tot2 * (1.0 / HIDDEN) - mean * mean
                inv = _rsqrt(var + EPS)
                for tt in range(L):
                    t = gb + tt
                    mv = _splat(mean, tt)
                    iv = _splat(inv, tt)
                    for e in range(NBLK):
                        t1 = gs[e] * iv
                        orws[t, pl.ds(L * e, L)] = (
                            (rows[t, pl.ds(L * e, L)] - mv) * t1 + bs[e])
                return c2

            lax.fori_loop(0, C // 2 // L, group_body, 0)

        def do_chunk(ci, slot, nxt):
            other = 1 - slot
            wait_p(ci, slot)
            @pl.when(ci >= 2)
            def _():
                wait_store(ci - 2, slot)

            mask_pass(ci, slot)
            ln_half(slot, 0)
            if nxt is not None:
                wait_w(ci + 1, other)
                issue_p(ci + 1, other)
            ln_half(slot, C // 2)
            issue_store(ci, slot)
            if nxt is not None:
                issue_w(nxt, slot)

        issue_w(0, 0)
        wait_w(0, 0)
        issue_p(0, 0)
        issue_w(1, 1)

        def pair_body(k, carry):
            i0 = 2 * k
            do_chunk(i0, 0, i0 + 2)
            do_chunk(i0 + 1, 1, i0 + 3)
            return carry

        lax.fori_loop(0, n_chunks // 2 - 1, pair_body, 0)

        nc = n_chunks
        wait_p(nc - 2, 0)
        wait_store(nc - 4, 0)
        mask_pass(nc - 2, 0)
        ln_half(0, 0)
        wait_w(nc - 1, 1)
        issue_p(nc - 1, 1)
        ln_half(0, C // 2)
        issue_store(nc - 2, 0)
        wait_p(nc - 1, 1)
        wait_store(nc - 3, 1)
        mask_pass(nc - 1, 1)
        ln_half(1, 0)
        ln_half(1, C // 2)
        issue_store(nc - 1, 1)
        wait_store(nc - 2, 0)
        wait_store(nc - 1, 1)

    return emb_kernel


@jax.jit
def _run(idw, idp, word_emb, pos_emb, ln_gamma, ln_beta):
    n_tokens = idw.shape[0]
    w = word_emb.at[PAD].set(0.0)
    p = pos_emb.at[PAD].set(0.0)
    return _make_kernel(n_tokens)(idw, idp, w, p, ln_gamma, ln_beta)


def kernel(uttr_ids_list, position_ids_list, word_emb, pos_emb, ln_gamma,
           ln_beta):
    B, S = uttr_ids_list.shape
    n = B * S
    out, mask = _run(uttr_ids_list.reshape(n), position_ids_list.reshape(n),
                     word_emb, pos_emb, ln_gamma, ln_beta)
    return out.reshape(B, S, HIDDEN), mask.reshape(B, S).astype(bool)


# speedup vs baseline: 1.5484x; 1.0853x over previous
import functools

import numpy as np

import jax
import jax.numpy as jnp
from jax import lax
from jax.experimental import pallas as pl
from jax.experimental.pallas import tpu as pltpu
from jax.experimental.pallas import tpu_sc as plsc

HIDDEN = 128
NPOS = 513
PAD = 0
EPS = 1e-5

NC = 2
NS = 16
NW = NC * NS
L = 16
NBLK = HIDDEN // L

C = 128

_RSQRT_MAGIC = np.int32(0x5F3759DF)

_GATHER_DNUMS = lax.GatherDimensionNumbers(
    offset_dims=(), collapsed_slice_dims=(0,), start_index_map=(0,))


def _splat(v, j):
    idx = jnp.full((L, 1), j, jnp.int32)
    return lax.gather(v, idx, _GATHER_DNUMS, (1,),
                      mode=lax.GatherScatterMode.PROMISE_IN_BOUNDS)


def _rsqrt(a):
    ai = lax.bitcast_convert_type(a, jnp.int32)
    y = lax.bitcast_convert_type(_RSQRT_MAGIC - (ai >> 1), jnp.float32)
    ha = a * 0.5
    for _ in range(3):
        y = y * (1.5 - ha * y * y)
    return y


def _make_kernel(n_tokens):
    assert n_tokens % (NW * C) == 0
    per_w = n_tokens // NW
    n_chunks = per_w // C
    assert n_chunks % 2 == 0 and n_chunks >= 4

    mesh = plsc.VectorSubcoreMesh(
        core_axis_name="c", subcore_axis_name="s",
        num_cores=NC, num_subcores=NS,
    )

    @functools.partial(
        pl.kernel,
        out_type=(
            jax.ShapeDtypeStruct((n_tokens, HIDDEN), jnp.float32),
            jax.ShapeDtypeStruct((n_tokens,), jnp.int32),
        ),
        mesh=mesh,
        compiler_params=pltpu.CompilerParams(needs_layout_passes=False),
        scratch_types=[
            pltpu.VMEM((per_w,), jnp.int32),
            pltpu.VMEM((per_w,), jnp.int32),
            pltpu.VMEM((2, C, HIDDEN), jnp.float32),
            pltpu.VMEM((2, C, HIDDEN), jnp.float32),
            pltpu.VMEM((17 * L,), jnp.float32),
            pltpu.VMEM((17 * L,), jnp.float32),
            pltpu.VMEM((2, C), jnp.int32),
            pltpu.VMEM_SHARED((NPOS, HIDDEN), jnp.float32),
            pltpu.VMEM((HIDDEN,), jnp.float32),
            pltpu.VMEM((HIDDEN,), jnp.float32),
            pltpu.SemaphoreType.DMA,
            pltpu.SemaphoreType.DMA,
            pltpu.SemaphoreType.DMA,
            pltpu.SemaphoreType.DMA,
            pltpu.SemaphoreType.DMA,
            pltpu.SemaphoreType.DMA,
        ],
    )
    def emb_kernel(idw_hbm, idp_hbm, wtab_hbm, ptab_hbm, g_hbm, b_hbm,
                   out_hbm, mask_hbm,
                   idw_v, idp_v, xrows, orows, msum, msq, mvec, ptab_s,
                   gv, bv,
                   semw0, semw1, semp0, semp1, semo0, semo1):
        wid = lax.axis_index("s") * NC + lax.axis_index("c")
        base = wid * per_w

        @pl.when(lax.axis_index("s") == 0)
        def _():
            pltpu.sync_copy(ptab_hbm, ptab_s)

        pltpu.sync_copy(idw_hbm.at[pl.ds(base, per_w)], idw_v)
        pltpu.sync_copy(idp_hbm.at[pl.ds(base, per_w)], idp_v)
        plsc.subcore_barrier()
        pltpu.sync_copy(g_hbm, gv)
        pltpu.sync_copy(b_hbm, bv)
        gs = [gv[pl.ds(L * e, L)] for e in range(NBLK)]
        bs = [bv[pl.ds(L * e, L)] for e in range(NBLK)]
        semw = (semw0, semw1)
        semp = (semp0, semp1)
        semo = (semo0, semo1)

        def issue_w(ci, slot):
            pltpu.async_copy(wtab_hbm.at[idw_v.at[pl.ds(ci * C, C)]],
                             xrows.at[slot], semw[slot])

        def wait_w(ci, slot):
            pltpu.make_async_copy(wtab_hbm.at[idw_v.at[pl.ds(ci * C, C)]],
                                  xrows.at[slot], semw[slot]).wait()

        def issue_p(ci, slot):
            pltpu.async_copy(ptab_s.at[idp_v.at[pl.ds(ci * C, C)]],
                             xrows.at[slot], semp[slot], add=True)

        def wait_p(ci, slot):
            pltpu.make_async_copy(ptab_s.at[idp_v.at[pl.ds(ci * C, C)]],
                                  xrows.at[slot], semp[slot]).wait()

        def issue_store(ci, slot):
            off = base + ci * C
            pltpu.async_copy(orows.at[slot], out_hbm.at[pl.ds(off, C)],
                             semo[slot])
            pltpu.async_copy(mvec.at[slot], mask_hbm.at[pl.ds(off, C)],
                             semo[slot])

        def wait_store(ci, slot):
            off = base + ci * C
            pltpu.make_async_copy(orows.at[slot],
                                  out_hbm.at[pl.ds(off, C)],
                                  semo[slot]).wait()
            pltpu.make_async_copy(mvec.at[slot],
                                  mask_hbm.at[pl.ds(off, C)],
                                  semo[slot]).wait()

        def mask_pass(ci, slot):
            mv = mvec.at[slot]

            def mask_body(g, c2):
                v = idw_v[pl.ds(ci * C + g * L, L)]
                mv[pl.ds(g * L, L)] = 1 - jnp.minimum(v, 1)
                return c2

            lax.fori_loop(0, C // L, mask_body, 0)

        def ln_half(slot, lo):
            rows = xrows.at[slot]
            orws = orows.at[slot]
            i16 = lax.iota(jnp.int32, L)
            i17 = (i16 << 4) + i16

            def group_body(g, c2):
                gb = lo + g * L
                for tt in range(L):
                    t = gb + tt
                    xs = [rows[t, pl.ds(L * e, L)] for e in range(NBLK)]
                    s = xs[0]
                    ssq = xs[0] * xs[0]
                    for e in range(1, NBLK):
                        s = s + xs[e]
                        ssq = ssq + xs[e] * xs[e]
                    msum[pl.ds(17 * tt, L)] = s
                    msq[pl.ds(17 * tt, L)] = ssq
                tot = plsc.load_gather(msum, [i17])
                tot2 = plsc.load_gather(msq, [i17])
                for l in range(1, L):
                    tot = tot + plsc.load_gather(msum, [i17 + l])
                    tot2 = tot2 + plsc.load_gather(msq, [i17 + l])
                mean = tot * (1.0 / HIDDEN)
                var = tot2 * (1.0 / HIDDEN) - mean * mean
                inv = _rsqrt(var + EPS)
                for tt in range(L):
                    t = gb + tt
                    mv = _splat(mean, tt)
                    iv = _splat(inv, tt)
                    for e in range(NBLK):
                        t1 = gs[e] * iv
                        orws[t, pl.ds(L * e, L)] = (
                            (rows[t, pl.ds(L * e, L)] - mv) * t1 + bs[e])
                return c2

            lax.fori_loop(0, C // 2 // L, group_body, 0)

        def do_chunk(ci, slot, nxt):
            other = 1 - slot
            wait_p(ci, slot)
            @pl.when(ci >= 2)
            def _():
                wait_store(ci - 2, slot)

            mask_pass(ci, slot)
            ln_half(slot, 0)
            if nxt is not None:
                wait_w(ci + 1, other)
                issue_p(ci + 1, other)
            ln_half(slot, C // 2)
            issue_store(ci, slot)
            if nxt is not None:
                issue_w(nxt, slot)

        issue_w(0, 0)
        wait_w(0, 0)
        issue_p(0, 0)
        issue_w(1, 1)

        def pair_body(k, carry):
            i0 = 2 * k
            do_chunk(i0, 0, i0 + 2)
            do_chunk(i0 + 1, 1, i0 + 3)
            return carry

        lax.fori_loop(0, n_chunks // 2 - 1, pair_body, 0)

        nc = n_chunks
        wait_p(nc - 2, 0)
        wait_store(nc - 4, 0)
        mask_pass(nc - 2, 0)
        ln_half(0, 0)
        wait_w(nc - 1, 1)
        issue_p(nc - 1, 1)
        ln_half(0, C // 2)
        issue_store(nc - 2, 0)
        wait_p(nc - 1, 1)
        wait_store(nc - 3, 1)
        mask_pass(nc - 1, 1)
        ln_half(1, 0)
        ln_half(1, C // 2)
        issue_store(nc - 1, 1)
        wait_store(nc - 2, 0)
        wait_store(nc - 1, 1)

    return emb_kernel


@jax.jit
def _run(idw, idp, word_emb, pos_emb, ln_gamma, ln_beta):
    n_tokens = idw.shape[0]
    w = word_emb.at[PAD].set(0.0)
    p = pos_emb.at[PAD].set(0.0)
    return _make_kernel(n_tokens)(idw, idp, w, p, ln_gamma, ln_beta)


def kernel(uttr_ids_list, position_ids_list, word_emb, pos_emb, ln_gamma,
           ln_beta):
    B, S = uttr_ids_list.shape
    n = B * S
    out, mask = _run(uttr_ids_list.reshape(n), position_ids_list.reshape(n),
                     word_emb, pos_emb, ln_gamma, ln_beta)
    return out.reshape(B, S, HIDDEN), mask.reshape(B, S).astype(bool)


# speedup vs baseline: 1.8305x; 1.1822x over previous
import functools

import numpy as np

import jax
import jax.numpy as jnp
from jax import lax
from jax.experimental import pallas as pl
from jax.experimental.pallas import tpu as pltpu
from jax.experimental.pallas import tpu_sc as plsc

HIDDEN = 128
NPOS = 513
PAD = 0
EPS = 1e-5

NC = 2
NS = 16
NW = NC * NS
L = 16
NBLK = HIDDEN // L

C = 64

_RSQRT_MAGIC = np.int32(0x5F3759DF)

_GATHER_DNUMS = lax.GatherDimensionNumbers(
    offset_dims=(), collapsed_slice_dims=(0,), start_index_map=(0,))


def _splat(v, j):
    idx = jnp.full((L, 1), j, jnp.int32)
    return lax.gather(v, idx, _GATHER_DNUMS, (1,),
                      mode=lax.GatherScatterMode.PROMISE_IN_BOUNDS)


def _rsqrt(a):
    ai = lax.bitcast_convert_type(a, jnp.int32)
    y = lax.bitcast_convert_type(_RSQRT_MAGIC - (ai >> 1), jnp.float32)
    ha = a * 0.5
    for _ in range(3):
        y = y * (1.5 - ha * y * y)
    return y


def _make_kernel(n_tokens):
    assert n_tokens % (NW * C) == 0
    per_w = n_tokens // NW
    n_chunks = per_w // C
    assert n_chunks % 2 == 0 and n_chunks >= 4

    mesh = plsc.VectorSubcoreMesh(
        core_axis_name="c", subcore_axis_name="s",
        num_cores=NC, num_subcores=NS,
    )

    @functools.partial(
        pl.kernel,
        out_type=(
            jax.ShapeDtypeStruct((n_tokens, HIDDEN), jnp.float32),
            jax.ShapeDtypeStruct((n_tokens,), jnp.int32),
        ),
        mesh=mesh,
        compiler_params=pltpu.CompilerParams(needs_layout_passes=False),
        scratch_types=[
            pltpu.VMEM((per_w,), jnp.int32),
            pltpu.VMEM((per_w,), jnp.int32),
            pltpu.VMEM((2, C, HIDDEN), jnp.float32),
            pltpu.VMEM((2, C, HIDDEN), jnp.float32),
            pltpu.VMEM((C, HIDDEN), jnp.float32),
            pltpu.VMEM((2, C, HIDDEN), jnp.float32),
            pltpu.VMEM((17 * L,), jnp.float32),
            pltpu.VMEM((17 * L,), jnp.float32),
            pltpu.VMEM((2, C), jnp.int32),
            pltpu.VMEM_SHARED((NPOS, HIDDEN), jnp.float32),
            pltpu.VMEM((HIDDEN,), jnp.float32),
            pltpu.VMEM((HIDDEN,), jnp.float32),
            pltpu.SemaphoreType.DMA,
            pltpu.SemaphoreType.DMA,
            pltpu.SemaphoreType.DMA,
            pltpu.SemaphoreType.DMA,
            pltpu.SemaphoreType.DMA,
            pltpu.SemaphoreType.DMA,
        ],
    )
    def emb_kernel(idw_hbm, idp_hbm, wtab_hbm, ptab_hbm, g_hbm, b_hbm,
                   out_hbm, mask_hbm,
                   idw_v, idp_v, wrows, prows, xbuf, orows, msum, msq,
                   mvec, ptab_s, gv, bv,
                   semw0, semw1, semp0, semp1, semo0, semo1):
        wid = lax.axis_index("s") * NC + lax.axis_index("c")
        base = wid * per_w

        @pl.when(lax.axis_index("s") == 0)
        def _():
            pltpu.sync_copy(ptab_hbm, ptab_s)

        pltpu.sync_copy(idw_hbm.at[pl.ds(base, per_w)], idw_v)
        pltpu.sync_copy(idp_hbm.at[pl.ds(base, per_w)], idp_v)
        pltpu.sync_copy(g_hbm, gv)
        pltpu.sync_copy(b_hbm, bv)
        plsc.subcore_barrier()
        gs = [gv[pl.ds(L * e, L)] for e in range(NBLK)]
        bs = [bv[pl.ds(L * e, L)] for e in range(NBLK)]
        semw = (semw0, semw1)
        semp = (semp0, semp1)
        semo = (semo0, semo1)

        def issue_gathers(ci, slot):
            pltpu.async_copy(wtab_hbm.at[idw_v.at[pl.ds(ci * C, C)]],
                             wrows.at[slot], semw[slot])
            pltpu.async_copy(ptab_s.at[idp_v.at[pl.ds(ci * C, C)]],
                             prows.at[slot], semp[slot])

        def wait_gathers(ci, slot):
            pltpu.make_async_copy(wtab_hbm.at[idw_v.at[pl.ds(ci * C, C)]],
                                  wrows.at[slot], semw[slot]).wait()
            pltpu.make_async_copy(ptab_s.at[idp_v.at[pl.ds(ci * C, C)]],
                                  prows.at[slot], semp[slot]).wait()

        def issue_store(ci, slot):
            off = base + ci * C
            pltpu.async_copy(orows.at[slot], out_hbm.at[pl.ds(off, C)],
                             semo[slot])
            pltpu.async_copy(mvec.at[slot], mask_hbm.at[pl.ds(off, C)],
                             semo[slot])

        def wait_store(ci, slot):
            off = base + ci * C
            pltpu.make_async_copy(orows.at[slot],
                                  out_hbm.at[pl.ds(off, C)],
                                  semo[slot]).wait()
            pltpu.make_async_copy(mvec.at[slot],
                                  mask_hbm.at[pl.ds(off, C)],
                                  semo[slot]).wait()

        def mask_pass(ci, slot):
            mv = mvec.at[slot]

            def mask_body(g, c2):
                v = idw_v[pl.ds(ci * C + g * L, L)]
                mv[pl.ds(g * L, L)] = 1 - jnp.minimum(v, 1)
                return c2

            lax.fori_loop(0, C // L, mask_body, 0)

        def ln_chunk(slot):
            rows = wrows.at[slot]
            rowsp = prows.at[slot]
            orws = orows.at[slot]
            i16 = lax.iota(jnp.int32, L)
            i17 = (i16 << 4) + i16

            def group_body(g, c2):
                gb = g * L
                for tt in range(L):
                    t = gb + tt
                    xs = []
                    for e in range(NBLK):
                        x = (rows[t, pl.ds(L * e, L)]
                             + rowsp[t, pl.ds(L * e, L)])
                        xbuf[t, pl.ds(L * e, L)] = x
                        xs.append(x)
                    s = xs[0]
                    ssq = xs[0] * xs[0]
                    for e in range(1, NBLK):
                        s = s + xs[e]
                        ssq = ssq + xs[e] * xs[e]
                    msum[pl.ds(17 * tt, L)] = s
                    msq[pl.ds(17 * tt, L)] = ssq
                tot = plsc.load_gather(msum, [i17])
                tot2 = plsc.load_gather(msq, [i17])
                for l in range(1, L):
                    tot = tot + plsc.load_gather(msum, [i17 + l])
                    tot2 = tot2 + plsc.load_gather(msq, [i17 + l])
                mean = tot * (1.0 / HIDDEN)
                var = tot2 * (1.0 / HIDDEN) - mean * mean
                inv = _rsqrt(var + EPS)
                for tt in range(L):
                    t = gb + tt
                    mv = _splat(mean, tt)
                    iv = _splat(inv, tt)
                    for e in range(NBLK):
                        t1 = gs[e] * iv
                        orws[t, pl.ds(L * e, L)] = (
                            (xbuf[t, pl.ds(L * e, L)] - mv) * t1 + bs[e])
                return c2

            lax.fori_loop(0, C // L, group_body, 0)

        def do_chunk(ci, slot, nxt):
            wait_gathers(ci, slot)

            @pl.when(ci >= 2)
            def _():
                wait_store(ci - 2, slot)

            mask_pass(ci, slot)
            ln_chunk(slot)
            issue_store(ci, slot)
            if nxt is not None:
                issue_gathers(nxt, slot)

        issue_gathers(0, 0)
        issue_gathers(1, 1)

        def pair_body(k, carry):
            i0 = 2 * k
            do_chunk(i0, 0, i0 + 2)
            do_chunk(i0 + 1, 1, i0 + 3)
            return carry

        lax.fori_loop(0, n_chunks // 2 - 1, pair_body, 0)

        nc = n_chunks
        do_chunk(nc - 2, 0, None)
        do_chunk(nc - 1, 1, None)
        wait_store(nc - 2, 0)
        wait_store(nc - 1, 1)

    return emb_kernel


@jax.jit
def _run(idw, idp, word_emb, pos_emb, ln_gamma, ln_beta):
    n_tokens = idw.shape[0]
    w = word_emb.at[PAD].set(0.0)
    p = pos_emb.at[PAD].set(0.0)
    return _make_kernel(n_tokens)(idw, idp, w, p, ln_gamma, ln_beta)


def kernel(uttr_ids_list, position_ids_list, word_emb, pos_emb, ln_gamma,
           ln_beta):
    B, S = uttr_ids_list.shape
    n = B * S
    out, mask = _run(uttr_ids_list.reshape(n), position_ids_list.reshape(n),
                     word_emb, pos_emb, ln_gamma, ln_beta)
    return out.reshape(B, S, HIDDEN), mask.reshape(B, S).astype(bool)
